# Initial kernel scaffold; baseline (speedup 1.0000x reference)
import jax
import jax.numpy as jnp
from jax.experimental import pallas as pl


def kernel(inputs, token_table, pos_table):
    raise NotImplementedError("write your pallas kernel here")


# speedup vs baseline: 1.0552x; 1.0552x over previous
import functools

import jax
import jax.numpy as jnp
from jax import lax
from jax.experimental import pallas as pl
from jax.experimental.pallas import tpu as pltpu
from jax.experimental.pallas import tpu_sc as plsc

_MAXLEN = 200
_EMBED = 32
_NW = 32
_CHUNK = 128


@functools.lru_cache(maxsize=None)
def _build(batch):
    total = batch * _MAXLEN
    rows_per_w = total // _NW
    nchunk = rows_per_w // _CHUNK
    mesh = plsc.VectorSubcoreMesh(core_axis_name="c", subcore_axis_name="s")

    @functools.partial(
        pl.kernel,
        mesh=mesh,
        out_type=jax.ShapeDtypeStruct((total, _EMBED), jnp.float32),
        scratch_types=[
            pltpu.VMEM((nchunk, _CHUNK), jnp.int32),
            pltpu.VMEM((2 * _MAXLEN, _EMBED), jnp.float32),
            pltpu.VMEM((_CHUNK, _EMBED), jnp.float32),
            pltpu.SemaphoreType.DMA,
        ],
        compiler_params=pltpu.CompilerParams(use_tc_tiling_on_sc=False),
    )
    def kern(idx_hbm, tok_hbm, pos_hbm, out_hbm, idx_v, pos_v, rows_v, sem):
        wid = lax.axis_index("s") * 2 + lax.axis_index("c")
        pltpu.sync_copy(idx_hbm.at[wid], idx_v)
        pltpu.sync_copy(pos_hbm, pos_v)

        def chunk_body(j, carry):
            pltpu.async_copy(tok_hbm.at[idx_v.at[j]], rows_v, sem).wait()
            tbase = (j * _CHUNK) % _MAXLEN

            def add_body(r, c):
                t = tbase + r
                rows_v[r, pl.ds(0, 16)] = rows_v[r, pl.ds(0, 16)] + pos_v[t, pl.ds(0, 16)]
                rows_v[r, pl.ds(16, 16)] = rows_v[r, pl.ds(16, 16)] + pos_v[t, pl.ds(16, 16)]
                return c

            lax.fori_loop(0, _CHUNK, add_body, 0)
            pltpu.sync_copy(rows_v, out_hbm.at[pl.ds(wid * rows_per_w + j * _CHUNK, _CHUNK)])
            return carry

        lax.fori_loop(0, nchunk, chunk_body, 0)

    return kern


def kernel(inputs, token_table, pos_table):
    batch, maxlen = inputs.shape
    idx = inputs.reshape(_NW, (batch * maxlen) // (_NW * _CHUNK), _CHUNK)
    idx = idx.astype(jnp.int32)
    pos2 = jnp.concatenate([pos_table, pos_table], axis=0)
    out = _build(batch)(idx, token_table, pos2)
    return out.reshape(batch, maxlen, _EMBED)


# speedup vs baseline: 1.4544x; 1.3783x over previous
import functools

import jax
import jax.numpy as jnp
from jax import lax
from jax.experimental import pallas as pl
from jax.experimental.pallas import tpu as pltpu
from jax.experimental.pallas import tpu_sc as plsc

_MAXLEN = 200
_EMBED = 32
_NW = 32
_HALF = 100
_NBUF = 4


@functools.lru_cache(maxsize=None)
def _build(batch):
    nseq_w = batch // _NW
    mesh = plsc.VectorSubcoreMesh(core_axis_name="c", subcore_axis_name="s")

    @functools.partial(
        pl.kernel,
        mesh=mesh,
        out_type=jax.ShapeDtypeStruct((batch, _MAXLEN, _EMBED), jnp.float32),
        scratch_types=[
            pltpu.VMEM((nseq_w, 2, _HALF), jnp.int32),
            pltpu.VMEM((_MAXLEN, _EMBED), jnp.float32),
        ]
        + [pltpu.VMEM((_MAXLEN, _EMBED), jnp.float32) for _ in range(_NBUF)]
        + [pltpu.SemaphoreType.DMA for _ in range(2 * _NBUF)],
        compiler_params=pltpu.CompilerParams(use_tc_tiling_on_sc=False),
    )
    def kern(idx_hbm, tok_hbm, pos_hbm, out_hbm, idx_v, pos_v, *bufs_sems):
        bufs = bufs_sems[:_NBUF]
        gsem = bufs_sems[_NBUF:2 * _NBUF]
        ssem = bufs_sems[2 * _NBUF:]
        wid = lax.axis_index("s") * 2 + lax.axis_index("c")
        seq0 = wid * nseq_w
        pltpu.sync_copy(idx_hbm.at[pl.ds(seq0, nseq_w)], idx_v)
        pltpu.sync_copy(pos_hbm, pos_v)

        def gather(s, b):
            pltpu.async_copy(
                tok_hbm.at[idx_v.at[s, 0]], bufs[b].at[pl.ds(0, _HALF)], gsem[b])
            pltpu.async_copy(
                tok_hbm.at[idx_v.at[s, 1]], bufs[b].at[pl.ds(_HALF, _HALF)], gsem[b])

        def gather_wait(s, b):
            pltpu.make_async_copy(
                tok_hbm.at[idx_v.at[s, 0]], bufs[b].at[pl.ds(0, _HALF)], gsem[b]).wait()
            pltpu.make_async_copy(
                tok_hbm.at[idx_v.at[s, 1]], bufs[b].at[pl.ds(_HALF, _HALF)], gsem[b]).wait()

        def store(s, b):
            pltpu.async_copy(bufs[b], out_hbm.at[seq0 + s], ssem[b])

        def store_wait(s, b):
            pltpu.make_async_copy(bufs[b], out_hbm.at[seq0 + s], ssem[b]).wait()

        for b in range(_NBUF):
            gather(b, b)

        def ring_body(jbase, carry):
            for b in range(_NBUF):
                s = jbase + b
                gather_wait(s, b)

                @plsc.parallel_loop(0, _MAXLEN, unroll=8)
                def add_body(r):
                    plsc.addupdate(
                        bufs[b].at[r, pl.ds(0, 16)], pos_v[r, pl.ds(0, 16)])
                    plsc.addupdate(
                        bufs[b].at[r, pl.ds(16, 16)], pos_v[r, pl.ds(16, 16)])

                store(s, b)

                @pl.when(s + _NBUF < nseq_w)
                def _():
                    store_wait(s, b)
                    gather(s + _NBUF, b)
            return carry

        lax.fori_loop(0, nseq_w // _NBUF, lambda i, c: ring_body(i * _NBUF, c), 0)
        for b in range(_NBUF):
            store_wait(nseq_w - _NBUF + b, b)

    return kern


def kernel(inputs, token_table, pos_table):
    batch, maxlen = inputs.shape
    idx = inputs.reshape(batch, 2, _HALF).astype(jnp.int32)
    return _build(batch)(idx, token_table, pos_table)
